# baseline (speedup 1.0000x reference)
import jax
import jax.numpy as jnp
from jax import lax
from jax.experimental import pallas as pl
from jax.experimental.pallas import tpu as pltpu
from jax.experimental.pallas import tpu_sc as plsc

_B, _N, _T = 8, 16384, 512
_EOS = 0.5
_NC, _NS = 2, 16
_RPC = _B // _NC
_ZBUF = 8192


def _sc_body(idxcat, px, py, tx, ty, mask_out, psum_out,
             idx_v, sidx_v, tidx_v,
             pxg_v, pyg_v, txg_v, tyg_v, ones_v, zbuf_v, acc_v,
             sem_i, sem_z):
    c = lax.axis_index("c")
    s = lax.axis_index("s")
    b = c * _RPC + s
    active = s < _RPC

    @pl.when(active)
    def _():
        h_i = pltpu.async_copy(idxcat.at[pl.ds(b * 2 * _T, 2 * _T)], idx_v, sem_i)

        ones16 = jnp.ones((16,), jnp.float32)
        zeros16 = jnp.zeros((16,), jnp.float32)
        for i in range(_ZBUF // 16):
            zbuf_v[pl.ds(i * 16, 16)] = zeros16
        for i in range(_T // 16):
            ones_v[pl.ds(i * 16, 16)] = ones16

        h_z = [pltpu.async_copy(zbuf_v,
                                mask_out.at[pl.ds(b * _N + k * _ZBUF, _ZBUF)],
                                sem_z)
               for k in range(_N // _ZBUF)]

        h_i.wait()

        for i in range(_T // 16):
            sidx_v[pl.ds(i * 16, 16)] = idx_v[pl.ds(i * 16, 16)] + b * _N
            tidx_v[pl.ds(i * 16, 16)] = idx_v[pl.ds(_T + i * 16, 16)] + b * _T

        h_g = [pltpu.async_copy(px.at[sidx_v], pxg_v, sem_i),
               pltpu.async_copy(py.at[sidx_v], pyg_v, sem_i),
               pltpu.async_copy(tx.at[tidx_v], txg_v, sem_i),
               pltpu.async_copy(ty.at[tidx_v], tyg_v, sem_i)]

        for h in h_z:
            h.wait()
        h_s = pltpu.async_copy(ones_v, mask_out.at[sidx_v], sem_z)

        for h in h_g:
            h.wait()
        acc = jnp.zeros((16,), jnp.float32)
        for pr, tr in ((pxg_v, txg_v), (pyg_v, tyg_v)):
            for i in range(_T // 16):
                d = pr[pl.ds(i * 16, 16)] - tr[pl.ds(i * 16, 16)]
                acc = acc + d * d
        acc_v[...] = acc
        h_p = pltpu.async_copy(acc_v, psum_out.at[pl.ds(b * 16, 16)], sem_i)
        h_p.wait()
        h_s.wait()


def _sc_call(src_idx, tgt_idx, pred_points, tgt_points):
    kfn = pl.kernel(
        _sc_body,
        out_type=[
            jax.ShapeDtypeStruct((_B * _N,), jnp.float32),
            jax.ShapeDtypeStruct((_B * 16,), jnp.float32),
        ],
        mesh=plsc.VectorSubcoreMesh(core_axis_name="c", subcore_axis_name="s"),
        compiler_params=pltpu.CompilerParams(use_tc_tiling_on_sc=False,
                                             needs_layout_passes=False),
        scratch_types=[
            pltpu.VMEM((2 * _T,), jnp.int32),
            pltpu.VMEM((_T,), jnp.int32),
            pltpu.VMEM((_T,), jnp.int32),
            pltpu.VMEM((_T,), jnp.float32),
            pltpu.VMEM((_T,), jnp.float32),
            pltpu.VMEM((_T,), jnp.float32),
            pltpu.VMEM((_T,), jnp.float32),
            pltpu.VMEM((_T,), jnp.float32),
            pltpu.VMEM((_ZBUF,), jnp.float32),
            pltpu.VMEM((16,), jnp.float32),
            pltpu.SemaphoreType.DMA,
            pltpu.SemaphoreType.DMA,
        ],
    )
    idxcat = jnp.concatenate(
        [src_idx[:, None, :], tgt_idx[:, None, :]], axis=1).reshape(-1)
    return kfn(idxcat,
               pred_points[:, :, 0].reshape(-1), pred_points[:, :, 1].reshape(-1),
               tgt_points[:, :, 0].reshape(-1), tgt_points[:, :, 1].reshape(-1))


_GRID = 8
_ROWS = _B * _N // 128
_BROW = _ROWS // _GRID


def _tc_body(x0_ref, x1_ref, m_ref, psum_ref, out_ref, smem):
    i = pl.program_id(0)

    @pl.when(i == 0)
    def _():
        smem[0] = 0.0
        smem[1] = 0.0

    a = x0_ref[...]
    b = x1_ref[...]
    m = m_ref[...]
    mx = jnp.maximum(a, b)
    lse = mx + jnp.log1p(jnp.exp(-jnp.abs(a - b)))
    contrib = jnp.where(m > 0.0, b - lse, _EOS * (a - lse))
    smem[0] += jnp.sum(contrib)
    smem[1] += jnp.sum(m)

    @pl.when(i == _GRID - 1)
    def _():
        s_p = smem[0]
        s_m = smem[1]
        w_sum = _EOS * (_B * _N) + (1.0 - _EOS) * s_m
        out_ref[0] = -s_p / w_sum
        out_ref[1] = jnp.sum(psum_ref[...]) / jnp.float32(_B * _T)


def _tc_call(x0, x1, mask, psum):
    spec = pl.BlockSpec((_BROW, 128), lambda i: (i, 0))
    return pl.pallas_call(
        _tc_body,
        grid=(_GRID,),
        in_specs=[spec, spec, spec,
                  pl.BlockSpec((1, 128), lambda i: (0, 0))],
        out_specs=pl.BlockSpec(memory_space=pltpu.SMEM),
        out_shape=jax.ShapeDtypeStruct((2,), jnp.float32),
        scratch_shapes=[pltpu.SMEM((2,), jnp.float32)],
    )(x0, x1, mask, psum)


def kernel(pred_logits, pred_points, tgt_points, tgt_labels, src_idx, tgt_idx):
    del tgt_labels
    x0 = pred_logits[:, :, 0].reshape(_ROWS, 128)
    x1 = pred_logits[:, :, 1].reshape(_ROWS, 128)
    mask, psum = _sc_call(src_idx, tgt_idx, pred_points, tgt_points)
    return _tc_call(x0, x1, mask.reshape(_ROWS, 128), psum.reshape(1, 128))


# speedup vs baseline: 1.0091x; 1.0091x over previous
import jax
import jax.numpy as jnp
from jax import lax
from jax.experimental import pallas as pl
from jax.experimental.pallas import tpu as pltpu
from jax.experimental.pallas import tpu_sc as plsc

_B, _N, _T = 8, 16384, 512
_EOS = 0.5
_NC, _NS = 2, 16
_RPC = _B // _NC
_ZBUF = 4096


def _sc_body(idxcat, px, py, tx, ty, mask_out, psum_out,
             idx_v, sidx_v, tidx_v,
             pxg_v, pyg_v, txg_v, tyg_v, ones_v, zbuf_v, acc_v,
             sem_i, sem_z):
    c = lax.axis_index("c")
    s = lax.axis_index("s")
    b = c * _RPC + s
    active = s < _RPC

    @pl.when(active)
    def _():
        h_i = pltpu.async_copy(idxcat.at[pl.ds(b * 2 * _T, 2 * _T)], idx_v, sem_i)

        ones16 = jnp.ones((16,), jnp.float32)
        zeros16 = jnp.zeros((16,), jnp.float32)
        for i in range(_ZBUF // 16):
            zbuf_v[pl.ds(i * 16, 16)] = zeros16
        for i in range(_T // 16):
            ones_v[pl.ds(i * 16, 16)] = ones16

        h_z = [pltpu.async_copy(zbuf_v,
                                mask_out.at[pl.ds(b * _N + k * _ZBUF, _ZBUF)],
                                sem_z)
               for k in range(_N // _ZBUF)]

        h_i.wait()

        for i in range(_T // 16):
            sidx_v[pl.ds(i * 16, 16)] = idx_v[pl.ds(i * 16, 16)] + b * _N
            tidx_v[pl.ds(i * 16, 16)] = idx_v[pl.ds(_T + i * 16, 16)] + b * _T

        h_g = [pltpu.async_copy(px.at[sidx_v], pxg_v, sem_i),
               pltpu.async_copy(py.at[sidx_v], pyg_v, sem_i),
               pltpu.async_copy(tx.at[tidx_v], txg_v, sem_i),
               pltpu.async_copy(ty.at[tidx_v], tyg_v, sem_i)]

        for h in h_z:
            h.wait()
        h_s = pltpu.async_copy(ones_v, mask_out.at[sidx_v], sem_z)

        for h in h_g:
            h.wait()
        acc = jnp.zeros((16,), jnp.float32)
        for pr, tr in ((pxg_v, txg_v), (pyg_v, tyg_v)):
            for i in range(_T // 16):
                d = pr[pl.ds(i * 16, 16)] - tr[pl.ds(i * 16, 16)]
                acc = acc + d * d
        acc_v[...] = acc
        h_p = pltpu.async_copy(acc_v, psum_out.at[pl.ds(b * 16, 16)], sem_i)
        h_p.wait()
        h_s.wait()


def _sc_call(src_idx, tgt_idx, pred_points, tgt_points):
    kfn = pl.kernel(
        _sc_body,
        out_type=[
            jax.ShapeDtypeStruct((_B * _N,), jnp.float32),
            jax.ShapeDtypeStruct((_B * 16,), jnp.float32),
        ],
        mesh=plsc.VectorSubcoreMesh(core_axis_name="c", subcore_axis_name="s"),
        compiler_params=pltpu.CompilerParams(use_tc_tiling_on_sc=False,
                                             needs_layout_passes=False),
        scratch_types=[
            pltpu.VMEM((2 * _T,), jnp.int32),
            pltpu.VMEM((_T,), jnp.int32),
            pltpu.VMEM((_T,), jnp.int32),
            pltpu.VMEM((_T,), jnp.float32),
            pltpu.VMEM((_T,), jnp.float32),
            pltpu.VMEM((_T,), jnp.float32),
            pltpu.VMEM((_T,), jnp.float32),
            pltpu.VMEM((_T,), jnp.float32),
            pltpu.VMEM((_ZBUF,), jnp.float32),
            pltpu.VMEM((16,), jnp.float32),
            pltpu.SemaphoreType.DMA,
            pltpu.SemaphoreType.DMA,
        ],
    )
    idxcat = jnp.concatenate(
        [src_idx[:, None, :], tgt_idx[:, None, :]], axis=1).reshape(-1)
    return kfn(idxcat,
               pred_points[:, :, 0].reshape(-1), pred_points[:, :, 1].reshape(-1),
               tgt_points[:, :, 0].reshape(-1), tgt_points[:, :, 1].reshape(-1))


_GRID = 8
_ROWS = _B * _N // 128
_BROW = _ROWS // _GRID


def _tc_body(x0_ref, x1_ref, m_ref, psum_ref, out_ref, smem):
    i = pl.program_id(0)

    @pl.when(i == 0)
    def _():
        smem[0] = 0.0
        smem[1] = 0.0

    a = x0_ref[...]
    b = x1_ref[...]
    m = m_ref[...]
    mx = jnp.maximum(a, b)
    lse = mx + jnp.log1p(jnp.exp(-jnp.abs(a - b)))
    contrib = jnp.where(m > 0.0, b - lse, _EOS * (a - lse))
    smem[0] += jnp.sum(contrib)
    smem[1] += jnp.sum(m)

    @pl.when(i == _GRID - 1)
    def _():
        s_p = smem[0]
        s_m = smem[1]
        w_sum = _EOS * (_B * _N) + (1.0 - _EOS) * s_m
        out_ref[0] = -s_p / w_sum
        out_ref[1] = jnp.sum(psum_ref[...]) / jnp.float32(_B * _T)


def _tc_call(x0, x1, mask, psum):
    spec = pl.BlockSpec((_BROW, 128), lambda i: (i, 0))
    return pl.pallas_call(
        _tc_body,
        grid=(_GRID,),
        in_specs=[spec, spec, spec,
                  pl.BlockSpec((1, 128), lambda i: (0, 0))],
        out_specs=pl.BlockSpec(memory_space=pltpu.SMEM),
        out_shape=jax.ShapeDtypeStruct((2,), jnp.float32),
        scratch_shapes=[pltpu.SMEM((2,), jnp.float32)],
    )(x0, x1, mask, psum)


def kernel(pred_logits, pred_points, tgt_points, tgt_labels, src_idx, tgt_idx):
    del tgt_labels
    x0 = pred_logits[:, :, 0].reshape(_ROWS, 128)
    x1 = pred_logits[:, :, 1].reshape(_ROWS, 128)
    mask, psum = _sc_call(src_idx, tgt_idx, pred_points, tgt_points)
    return _tc_call(x0, x1, mask.reshape(_ROWS, 128), psum.reshape(1, 128))
